# Initial kernel scaffold; baseline (speedup 1.0000x reference)
import jax
import jax.numpy as jnp
from jax.experimental import pallas as pl


def kernel(z, codebook):
    raise NotImplementedError("write your pallas kernel here")


# speedup vs baseline: 1.0538x; 1.0538x over previous
import functools

import jax
import jax.numpy as jnp
from jax import lax
from jax.experimental import pallas as pl
from jax.experimental.pallas import tpu as pltpu

_CODE_DIM = 64
_NUM_CODES = 1024
_COMMIT = 0.25


def _vq_body(z_ref, cb_ref, zq_ref, idx_ref, loss_ref, perp_ref,
             acc_ref, counts_ref, *, num_batches, hw, n_total):
    b = pl.program_id(0)
    zb = z_ref[0]
    cb = cb_ref[...]

    csum = jnp.sum(cb * cb, axis=1, keepdims=True)
    zsq = jnp.sum(zb * zb, axis=0, keepdims=True)

    mm = lax.dot_general(cb, zb, (((1,), (0,)), ((), ())),
                         preferred_element_type=jnp.float32)
    distT = (zsq + csum) - 2.0 * mm

    minv = jnp.min(distT, axis=0, keepdims=True)
    kio = lax.broadcasted_iota(jnp.int32, (_NUM_CODES, hw), 0)
    idx = jnp.min(jnp.where(distT == minv, kio, _NUM_CODES), axis=0,
                  keepdims=True)
    idx_ref[0] = idx.astype(jnp.int32)

    oh = jnp.where(kio == idx, 1.0, 0.0).astype(jnp.float32)
    zq = lax.dot_general(cb, oh, (((0,), (0,)), ((), ())),
                         preferred_element_type=jnp.float32,
                         precision=lax.Precision.HIGHEST)
    zq_ref[0] = zq

    @pl.when(b == 0)
    def _init():
        acc_ref[0, 0] = 0.0
        counts_ref[...] = jnp.zeros_like(counts_ref)

    acc_ref[0, 0] += jnp.sum(minv)
    counts_ref[...] += jnp.sum(oh, axis=1, keepdims=True)

    @pl.when(b == num_batches - 1)
    def _finish():
        loss_ref[0, 0] = (1.0 + _COMMIT) * acc_ref[0, 0] / float(n_total * _CODE_DIM)
        avg = counts_ref[...] * (1.0 / float(n_total))
        ent = -jnp.sum(avg * jnp.log(avg + 1e-10))
        perp_ref[0, 0] = jnp.exp(ent)


def kernel(z, codebook):
    B, C, H, W = z.shape
    hw = H * W
    n_total = B * hw
    z3 = z.reshape(B, C, hw)

    grid = (B,)
    zq3, idx3, loss2, perp2 = pl.pallas_call(
        functools.partial(_vq_body, num_batches=B, hw=hw, n_total=n_total),
        grid=grid,
        in_specs=[
            pl.BlockSpec((1, C, hw), lambda b: (b, 0, 0)),
            pl.BlockSpec((_NUM_CODES, _CODE_DIM), lambda b: (0, 0)),
        ],
        out_specs=[
            pl.BlockSpec((1, C, hw), lambda b: (b, 0, 0)),
            pl.BlockSpec((1, 1, hw), lambda b: (b, 0, 0)),
            pl.BlockSpec(memory_space=pltpu.SMEM, block_shape=(1, 1),
                         index_map=lambda b: (0, 0)),
            pl.BlockSpec(memory_space=pltpu.SMEM, block_shape=(1, 1),
                         index_map=lambda b: (0, 0)),
        ],
        out_shape=[
            jax.ShapeDtypeStruct((B, C, hw), jnp.float32),
            jax.ShapeDtypeStruct((B, 1, hw), jnp.int32),
            jax.ShapeDtypeStruct((1, 1), jnp.float32),
            jax.ShapeDtypeStruct((1, 1), jnp.float32),
        ],
        scratch_shapes=[
            pltpu.SMEM((1, 1), jnp.float32),
            pltpu.VMEM((_NUM_CODES, 1), jnp.float32),
        ],
    )(z3, codebook)

    z_q = zq3.reshape(B, C, H, W)
    indices = idx3.reshape(B, H, W)
    return (z_q, loss2[0, 0], (indices, perp2[0, 0]))


# speedup vs baseline: 1.5921x; 1.5109x over previous
import functools

import jax
import jax.numpy as jnp
from jax import lax
from jax.experimental import pallas as pl
from jax.experimental.pallas import tpu as pltpu
from jax.experimental.pallas import tpu_sc as plsc

_CODE_DIM = 64
_NUM_CODES = 1024
_COMMIT = 0.25
_NBLK = 2304


def _vq_body(z_ref, cb_ref, idx_ref, loss_ref, perp_ref,
             acc_ref, counts_ref, *, num_steps, n_total):
    i = pl.program_id(0)
    zb = z_ref[...]
    cb = cb_ref[...]

    csum = jnp.sum(cb * cb, axis=1, keepdims=True)
    zsq = jnp.sum(zb * zb, axis=0, keepdims=True)

    mm = lax.dot_general(cb, zb, (((1,), (0,)), ((), ())),
                         preferred_element_type=jnp.float32)
    distT = (zsq + csum) - 2.0 * mm

    minv = jnp.min(distT, axis=0, keepdims=True)
    kio = lax.broadcasted_iota(jnp.int32, (_NUM_CODES, _NBLK), 0)
    hit = distT == minv
    idx = jnp.min(jnp.where(hit, kio, _NUM_CODES), axis=0,
                  keepdims=True)
    idx_ref[0] = idx.astype(jnp.int32)

    @pl.when(i == 0)
    def _init():
        acc_ref[0, 0] = 0.0
        counts_ref[...] = jnp.zeros_like(counts_ref)

    acc_ref[0, 0] += jnp.sum(minv)
    oh = jnp.where(kio == idx, 1.0, 0.0).astype(jnp.float32)
    counts_ref[...] += jnp.sum(oh, axis=1, keepdims=True)

    @pl.when(i == num_steps - 1)
    def _finish():
        loss_ref[0, 0] = (1.0 + _COMMIT) * acc_ref[0, 0] / float(n_total * _CODE_DIM)
        avg = counts_ref[...] * (1.0 / float(n_total))
        ent = -jnp.sum(avg * jnp.log(avg + 1e-10))
        perp_ref[0, 0] = jnp.exp(ent)


def _tc_stage(zT, codebook, n_total):
    num_steps = n_total // _NBLK
    return pl.pallas_call(
        functools.partial(_vq_body, num_steps=num_steps, n_total=n_total),
        grid=(num_steps,),
        in_specs=[
            pl.BlockSpec((_CODE_DIM, _NBLK), lambda i: (0, i)),
            pl.BlockSpec((_NUM_CODES, _CODE_DIM), lambda i: (0, 0)),
        ],
        out_specs=[
            pl.BlockSpec((1, 1, _NBLK), lambda i: (i, 0, 0)),
            pl.BlockSpec(memory_space=pltpu.SMEM, block_shape=(1, 1),
                         index_map=lambda i: (0, 0)),
            pl.BlockSpec(memory_space=pltpu.SMEM, block_shape=(1, 1),
                         index_map=lambda i: (0, 0)),
        ],
        out_shape=[
            jax.ShapeDtypeStruct((num_steps, 1, _NBLK), jnp.int32),
            jax.ShapeDtypeStruct((1, 1), jnp.float32),
            jax.ShapeDtypeStruct((1, 1), jnp.float32),
        ],
        scratch_shapes=[
            pltpu.SMEM((1, 1), jnp.float32),
            pltpu.VMEM((_NUM_CODES, 1), jnp.float32),
        ],
    )(zT, codebook)


def _sc_gather(cb128, idx_flat, n_total):
    info = plsc.get_sparse_core_info()
    nw = info.num_cores * info.num_subcores
    b_per_w = n_total // nw
    mesh = plsc.VectorSubcoreMesh(core_axis_name="c", subcore_axis_name="s")

    @functools.partial(
        pl.kernel, mesh=mesh,
        out_type=jax.ShapeDtypeStruct((n_total, 128), jnp.float32),
        scratch_types=[
            pltpu.VMEM((b_per_w,), jnp.int32),
            pltpu.VMEM((b_per_w, 128), jnp.float32),
            pltpu.SemaphoreType.DMA,
        ],
    )
    def gather(cb_hbm, idx_hbm, out_hbm, idx_v, rows_v, sem):
        wid = lax.axis_index("s") * info.num_cores + lax.axis_index("c")
        base = wid * b_per_w
        pltpu.sync_copy(idx_hbm.at[pl.ds(base, b_per_w)], idx_v)
        pltpu.async_copy(cb_hbm.at[idx_v], rows_v, sem).wait()
        pltpu.sync_copy(rows_v, out_hbm.at[pl.ds(base, b_per_w)])

    return gather(cb128, idx_flat)


def kernel(z, codebook):
    B, C, H, W = z.shape
    hw = H * W
    n_total = B * hw
    zT = z.reshape(B, C, hw).transpose(1, 0, 2).reshape(C, n_total)

    idx3, loss2, perp2 = _tc_stage(zT, codebook, n_total)
    idx_flat = idx3.reshape(n_total)

    cb128 = jnp.pad(codebook, ((0, 0), (0, 128 - _CODE_DIM)))
    rows = _sc_gather(cb128, idx_flat, n_total)
    z_q = rows[:, :_CODE_DIM].reshape(B, hw, C).transpose(0, 2, 1).reshape(B, C, H, W)

    indices = idx_flat.reshape(B, H, W)
    return (z_q, loss2[0, 0], (indices, perp2[0, 0]))


# speedup vs baseline: 1.9910x; 1.2505x over previous
import functools

import jax
import jax.numpy as jnp
from jax import lax
from jax.experimental import pallas as pl
from jax.experimental.pallas import tpu as pltpu

_CODE_DIM = 64
_NUM_CODES = 1024
_COMMIT = 0.25
_NBLK = 2304


def _vq_body(z_ref, cb_ref, cbm2_ref, idx_ref, zq_ref, loss_ref, perp_ref,
             acc_ref, counts_ref, *, num_steps, n_total):
    i = pl.program_id(0)
    zb = z_ref[...]
    cb = cb_ref[...]
    cbm2 = cbm2_ref[...]

    csum = jnp.sum(cb * cb, axis=1, keepdims=True)
    zsq = jnp.sum(zb * zb, axis=0, keepdims=True)

    mm2 = lax.dot_general(cbm2, zb, (((1,), (0,)), ((), ())),
                          preferred_element_type=jnp.float32)
    distT = (zsq + csum) + mm2

    minv = jnp.min(distT, axis=0, keepdims=True)
    kio = lax.broadcasted_iota(jnp.int32, (_NUM_CODES, _NBLK), 0)
    hit = distT == minv
    idx = jnp.min(jnp.where(hit, kio, _NUM_CODES), axis=0,
                  keepdims=True)
    idx_ref[0] = idx.astype(jnp.int32)

    oh = jnp.where(kio == idx, 1.0, 0.0).astype(jnp.float32)
    zq_ref[...] = lax.dot_general(cb, oh, (((0,), (0,)), ((), ())),
                                  preferred_element_type=jnp.float32)

    @pl.when(i == 0)
    def _init():
        acc_ref[0, 0] = 0.0
        counts_ref[...] = jnp.zeros_like(counts_ref)

    acc_ref[0, 0] += jnp.sum(minv)
    counts_ref[...] += jnp.sum(oh, axis=1, keepdims=True)

    @pl.when(i == num_steps - 1)
    def _finish():
        loss_ref[0, 0] = (1.0 + _COMMIT) * acc_ref[0, 0] / float(n_total * _CODE_DIM)
        avg = counts_ref[...] * (1.0 / float(n_total))
        ent = -jnp.sum(avg * jnp.log(avg + 1e-10))
        perp_ref[0, 0] = jnp.exp(ent)


def kernel(z, codebook):
    B, C, H, W = z.shape
    hw = H * W
    n_total = B * hw
    num_steps = n_total // _NBLK
    zT = z.reshape(B, C, hw).transpose(1, 0, 2).reshape(C, n_total)
    cbm2 = -2.0 * codebook

    idx3, zqT, loss2, perp2 = pl.pallas_call(
        functools.partial(_vq_body, num_steps=num_steps, n_total=n_total),
        grid=(num_steps,),
        in_specs=[
            pl.BlockSpec((_CODE_DIM, _NBLK), lambda i: (0, i)),
            pl.BlockSpec((_NUM_CODES, _CODE_DIM), lambda i: (0, 0)),
            pl.BlockSpec((_NUM_CODES, _CODE_DIM), lambda i: (0, 0)),
        ],
        out_specs=[
            pl.BlockSpec((1, 1, _NBLK), lambda i: (i, 0, 0)),
            pl.BlockSpec((_CODE_DIM, _NBLK), lambda i: (0, i)),
            pl.BlockSpec(memory_space=pltpu.SMEM, block_shape=(1, 1),
                         index_map=lambda i: (0, 0)),
            pl.BlockSpec(memory_space=pltpu.SMEM, block_shape=(1, 1),
                         index_map=lambda i: (0, 0)),
        ],
        out_shape=[
            jax.ShapeDtypeStruct((num_steps, 1, _NBLK), jnp.int32),
            jax.ShapeDtypeStruct((_CODE_DIM, n_total), jnp.float32),
            jax.ShapeDtypeStruct((1, 1), jnp.float32),
            jax.ShapeDtypeStruct((1, 1), jnp.float32),
        ],
        scratch_shapes=[
            pltpu.SMEM((1, 1), jnp.float32),
            pltpu.VMEM((_NUM_CODES, 1), jnp.float32),
        ],
    )(zT, codebook, cbm2)

    z_q = zqT.reshape(C, B, hw).transpose(1, 0, 2).reshape(B, C, H, W)
    indices = idx3.reshape(B, H, W)
    return (z_q, loss2[0, 0], (indices, perp2[0, 0]))


# speedup vs baseline: 2.0852x; 1.0473x over previous
import functools

import jax
import jax.numpy as jnp
from jax import lax
from jax.experimental import pallas as pl
from jax.experimental.pallas import tpu as pltpu

_CODE_DIM = 64
_NUM_CODES = 1024
_COMMIT = 0.25
_BB = 4
_HW = 576
_NBLK = _BB * _HW


def _vq_body(z_ref, cb_ref, cbm2_ref, idx_ref, zq_ref, loss_ref, perp_ref,
             acc_ref, counts_ref, *, num_steps, n_total):
    i = pl.program_id(0)
    zb = jnp.concatenate([z_ref[s] for s in range(_BB)], axis=1)
    cb = cb_ref[...]
    cbm2 = cbm2_ref[...]

    csum = jnp.sum(cb * cb, axis=1, keepdims=True)
    zsq = jnp.sum(zb * zb, axis=0, keepdims=True)

    mm2 = lax.dot_general(cbm2, zb, (((1,), (0,)), ((), ())),
                          preferred_element_type=jnp.float32)
    distT = (zsq + csum) + mm2

    minv = jnp.min(distT, axis=0, keepdims=True)
    kio = lax.broadcasted_iota(jnp.int32, (_NUM_CODES, _NBLK), 0)
    hit = distT == minv
    idx = jnp.min(jnp.where(hit, kio, _NUM_CODES), axis=0,
                  keepdims=True)
    for s in range(_BB):
        idx_ref[s] = idx[:, s * _HW:(s + 1) * _HW].astype(jnp.int32)

    oh = jnp.where(kio == idx, 1.0, 0.0).astype(jnp.float32)
    zq = lax.dot_general(cb, oh, (((0,), (0,)), ((), ())),
                         preferred_element_type=jnp.float32)
    for s in range(_BB):
        zq_ref[s] = zq[:, s * _HW:(s + 1) * _HW]

    @pl.when(i == 0)
    def _init():
        acc_ref[0, 0] = 0.0
        counts_ref[...] = jnp.zeros_like(counts_ref)

    acc_ref[0, 0] += jnp.sum(minv)
    counts_ref[...] += jnp.sum(oh, axis=1, keepdims=True)

    @pl.when(i == num_steps - 1)
    def _finish():
        loss_ref[0, 0] = (1.0 + _COMMIT) * acc_ref[0, 0] / float(n_total * _CODE_DIM)
        avg = counts_ref[...] * (1.0 / float(n_total))
        ent = -jnp.sum(avg * jnp.log(avg + 1e-10))
        perp_ref[0, 0] = jnp.exp(ent)


def kernel(z, codebook):
    B, C, H, W = z.shape
    hw = H * W
    n_total = B * hw
    num_steps = B // _BB
    z3 = z.reshape(B, C, hw)
    cbm2 = -2.0 * codebook

    idx3, zq3, loss2, perp2 = pl.pallas_call(
        functools.partial(_vq_body, num_steps=num_steps, n_total=n_total),
        grid=(num_steps,),
        in_specs=[
            pl.BlockSpec((_BB, C, hw), lambda i: (i, 0, 0)),
            pl.BlockSpec((_NUM_CODES, _CODE_DIM), lambda i: (0, 0)),
            pl.BlockSpec((_NUM_CODES, _CODE_DIM), lambda i: (0, 0)),
        ],
        out_specs=[
            pl.BlockSpec((_BB, 1, hw), lambda i: (i, 0, 0)),
            pl.BlockSpec((_BB, C, hw), lambda i: (i, 0, 0)),
            pl.BlockSpec(memory_space=pltpu.SMEM, block_shape=(1, 1),
                         index_map=lambda i: (0, 0)),
            pl.BlockSpec(memory_space=pltpu.SMEM, block_shape=(1, 1),
                         index_map=lambda i: (0, 0)),
        ],
        out_shape=[
            jax.ShapeDtypeStruct((B, 1, hw), jnp.int32),
            jax.ShapeDtypeStruct((B, C, hw), jnp.float32),
            jax.ShapeDtypeStruct((1, 1), jnp.float32),
            jax.ShapeDtypeStruct((1, 1), jnp.float32),
        ],
        scratch_shapes=[
            pltpu.SMEM((1, 1), jnp.float32),
            pltpu.VMEM((_NUM_CODES, 1), jnp.float32),
        ],
    )(z3, codebook, cbm2)

    z_q = zq3.reshape(B, C, H, W)
    indices = idx3.reshape(B, H, W)
    return (z_q, loss2[0, 0], (indices, perp2[0, 0]))
